# speedup vs baseline: 9.0595x; 9.0595x over previous
import functools

import jax
import jax.numpy as jnp
from jax.experimental import pallas as pl
from jax.experimental.pallas import tpu as pltpu

_CLS = 32000
_SMOOTH = 0.1
_N = 2048
_R = 128
_NB = _N // _R
_CH = 3200


def _tc_body(x_ref, t_ref, o_ref, acc_ref):
    i = pl.program_id(0)
    a = _SMOOTH / (_CLS - 1)
    b = 1.0 - _SMOOTH - a

    t = t_ref[0]

    m = jnp.max(x_ref[:, 0:_CH], axis=1, keepdims=True)
    for c in range(_CH, _CLS, _CH):
        m = jnp.maximum(m, jnp.max(x_ref[:, c:c + _CH], axis=1, keepdims=True))

    se = jnp.zeros((_R, 1), jnp.float32)
    s = jnp.zeros((_R, 1), jnp.float32)
    pt = jnp.zeros((_R, 1), jnp.float32)
    for c in range(0, _CLS, _CH):
        x = x_ref[:, c:c + _CH]
        se = se + jnp.sum(jnp.exp(x - m), axis=1, keepdims=True)
        s = s + jnp.sum(x, axis=1, keepdims=True)
        cols = jax.lax.broadcasted_iota(jnp.int32, (_R, _CH), 1) + c
        pt = pt + jnp.sum(jnp.where(cols == t, x, 0.0), axis=1, keepdims=True)

    lse = m + jnp.log(se)
    partial = jnp.sum(lse - a * s - b * pt).reshape(1, 1)

    @pl.when(i == 0)
    def _():
        acc_ref[:, :] = jnp.zeros((1, 1), jnp.float32)

    acc_ref[:, :] = acc_ref[:, :] + partial

    @pl.when(i == _NB - 1)
    def _():
        o_ref[:, :] = acc_ref[:, :] * (1.0 / _N)


@jax.jit
def kernel(pred, target):
    t3 = target.astype(jnp.int32).reshape(_NB, _R, 1)
    out = pl.pallas_call(
        _tc_body,
        grid=(_NB,),
        in_specs=[
            pl.BlockSpec((_R, _CLS), lambda i: (i, 0)),
            pl.BlockSpec((1, _R, 1), lambda i: (i, 0, 0)),
        ],
        out_specs=pl.BlockSpec((1, 1), lambda i: (0, 0)),
        out_shape=jax.ShapeDtypeStruct((1, 1), jnp.float32),
        scratch_shapes=[pltpu.VMEM((1, 1), jnp.float32)],
    )(pred, t3)
    return out[0, 0]
